# Initial kernel scaffold; baseline (speedup 1.0000x reference)
import jax
import jax.numpy as jnp
from jax.experimental import pallas as pl


def kernel(x_t, idx_C, idx_A, idx_T, emb_C, emb_A, emb_T):
    raise NotImplementedError("write your pallas kernel here")


# speedup vs baseline: 1.0452x; 1.0452x over previous
import functools

import jax
import jax.numpy as jnp
from jax.experimental import pallas as pl
from jax.experimental.pallas import tpu as pltpu

BB = 8


def _fold_kernel(idx_c_ref, idx_a_ref, idx_t_ref,
                 emb_c_ref, emb_a_ref, emb_t_ref, out_ref, *, n):
    i = pl.program_id(0)
    b0 = i * BB
    for j in range(BB):
        c = idx_c_ref[b0 + j]
        a = idx_a_ref[b0 + j]
        t = idx_t_ref[b0 + j]
        row_c = emb_c_ref[pl.ds(c, 1), :]
        row_a = emb_a_ref[pl.ds(a, 1), :]
        row_t = emb_t_ref[pl.ds(t, 1), :]
        row = jnp.concatenate([row_c, row_a, row_t], axis=-1)
        out_ref[j, :, :] = jnp.broadcast_to(row, (n, row.shape[-1]))


def kernel(x_t, idx_C, idx_A, idx_T, emb_C, emb_A, emb_T):
    b, n = x_t.shape[0], x_t.shape[1]
    d3 = emb_C.shape[1] + emb_A.shape[1] + emb_T.shape[1]
    grid = b // BB
    out = pl.pallas_call(
        functools.partial(_fold_kernel, n=n),
        grid_spec=pltpu.PrefetchScalarGridSpec(
            num_scalar_prefetch=3,
            grid=(grid,),
            in_specs=[
                pl.BlockSpec(emb_C.shape, lambda i, *_: (0, 0)),
                pl.BlockSpec(emb_A.shape, lambda i, *_: (0, 0)),
                pl.BlockSpec(emb_T.shape, lambda i, *_: (0, 0)),
            ],
            out_specs=pl.BlockSpec((BB, n, d3), lambda i, *_: (i, 0, 0)),
        ),
        out_shape=jax.ShapeDtypeStruct((b, n, d3), jnp.float32),
    )(idx_C.astype(jnp.int32), idx_A.astype(jnp.int32),
      idx_T.astype(jnp.int32), emb_C, emb_A, emb_T)
    return out
